# baseline (speedup 1.0000x reference)
import functools

import jax
import jax.numpy as jnp
from jax import lax
from jax.experimental import pallas as pl
from jax.experimental.pallas import tpu as pltpu
from jax.experimental.pallas import tpu_sc as plsc

_M = 26
_V = 100001
_K = 16
_B = 4096
_H = 400
_L = 2
_EPS = 1e-5

_NC = 2
_NS = 16
_NW = _NC * _NS
_RPW = _B * _M // _NW
_CPW = _RPW // 128
_SPW = _B // _NW


def _sc_gather(x3d, offs, o2t, o1p):
    mesh = plsc.VectorSubcoreMesh(core_axis_name="c", subcore_axis_name="s")

    @functools.partial(
        pl.kernel,
        out_type=(
            jax.ShapeDtypeStruct((_B * _M, _K), jnp.float32),
            jax.ShapeDtypeStruct((_B * _M, 16), jnp.float32),
        ),
        mesh=mesh,
        compiler_params=pltpu.CompilerParams(use_tc_tiling_on_sc=False),
        scratch_types=[
            pltpu.VMEM((_CPW, 128), jnp.int32),
            pltpu.VMEM((_CPW, 128), jnp.int32),
            pltpu.VMEM((_CPW, 128), jnp.int32),
            pltpu.VMEM((_CPW, 128), jnp.int32),
            pltpu.VMEM((_RPW, _K), jnp.float32),
            pltpu.VMEM((_RPW, 16), jnp.float32),
            pltpu.SemaphoreType.DMA,
            pltpu.SemaphoreType.DMA,
        ],
    )
    def k(x3d_h, offs_h, o2t_h, o1p_h, e2_h, e1_h,
          xv, ov, iv, bv, rv, e1v, s2, s1):
        w = lax.axis_index("s") * _NC + lax.axis_index("c")
        pltpu.sync_copy(x3d_h.at[w], xv)
        pltpu.sync_copy(offs_h, ov)
        for j in range(_CPW):
            for t in range(8):
                sl = pl.ds(t * 16, 16)
                ids = xv[j, sl] + ov[j, sl]
                iv[j, sl] = ids
                bv[j, sl] = lax.shift_right_logical(ids, 4)
        cps = []
        for j in range(_CPW):
            cps.append(pltpu.async_copy(
                o2t_h.at[iv.at[j]], rv.at[pl.ds(j * 128, 128)], s2))
            cps.append(pltpu.async_copy(
                o1p_h.at[bv.at[j]], e1v.at[pl.ds(j * 128, 128)], s1))
        for cp in cps:
            cp.wait()
        pltpu.sync_copy(rv, e2_h.at[pl.ds(w * _RPW, _RPW)])
        pltpu.sync_copy(e1v, e1_h.at[pl.ds(w * _RPW, _RPW)])

    return k(x3d, offs, o2t, o1p)


def _tc_body(ef_ref, e1_ref, oh_ref, sel_ref, bias_ref, w1_ref, b1_ref, g1_ref,
             bt1_ref, ws_ref, bs_ref, gs_ref, bts_ref, wf_ref, bf_ref,
             out_ref):
    f32 = jnp.float32
    hi = lax.Precision.DEFAULT
    ef = ef_ref[...]
    fm1 = jnp.sum((e1_ref[...] * oh_ref[...]).astype(f32), axis=1,
                  keepdims=True)
    sel = sel_ref[...]
    s = lax.dot_general(ef, sel, (((1,), (0,)), ((), ())),
                        precision=hi, preferred_element_type=f32)
    t2 = lax.dot_general(ef * ef, sel, (((1,), (0,)), ((), ())),
                         precision=hi, preferred_element_type=f32)
    fm2 = 0.5 * jnp.sum(s * s - t2, axis=1, keepdims=True)
    y = bias_ref[...] + fm1 + fm2

    def bn_relu(h, g, b):
        mu = jnp.mean(h, axis=0, keepdims=True)
        hc = h - mu
        var = jnp.mean(hc * hc, axis=0, keepdims=True)
        return jnp.maximum(hc * lax.rsqrt(var + _EPS) * g + b, 0.0)

    h = lax.dot_general(ef, w1_ref[...], (((1,), (1,)), ((), ())),
                        precision=hi, preferred_element_type=f32)
    h = bn_relu(h + b1_ref[...], g1_ref[...], bt1_ref[...])
    for i in range(_L):
        h = lax.dot_general(h, ws_ref[i], (((1,), (1,)), ((), ())),
                            precision=hi, preferred_element_type=f32)
        h = bn_relu(h + bs_ref[i], gs_ref[i], bts_ref[i])
    ydnn = lax.dot_general(h, wf_ref[...], (((1,), (1,)), ((), ())),
                           precision=hi, preferred_element_type=f32)
    out_ref[...] = y + ydnn + bf_ref[...]


def kernel(x, o1, o2, bias, W1, b1, g1, bt1, Ws, bs, gs, bts, Wf, bf):
    x3d = x.astype(jnp.int32).reshape(_NW, _CPW, 128)
    offs = (jnp.arange(_RPW, dtype=jnp.int32) % _M * _V).reshape(_CPW, 128)
    o2t = o2.reshape(_M * _V, _K)
    o1p = jnp.pad(o1.reshape(_M * _V), (0, (-_M * _V) % 16)).reshape(-1, 16)
    e2r, e1r = _sc_gather(x3d, offs, o2t, o1p)
    ef = e2r.reshape(_B, _M * _K)
    e1m = e1r.reshape(_B, _M * 16).astype(jnp.bfloat16)
    flat = x.astype(jnp.int32) + jnp.arange(_M, dtype=jnp.int32)[None, :] * _V
    oh = (jnp.bitwise_and(flat, 15)[..., None]
          == jnp.arange(16, dtype=jnp.int32)).astype(jnp.bfloat16)
    oh = oh.reshape(_B, _M * 16)

    sel = (jnp.arange(_M * _K, dtype=jnp.int32)[:, None] % _K
           == jnp.arange(_K, dtype=jnp.int32)[None, :]).astype(jnp.float32)
    out = pl.pallas_call(
        _tc_body,
        out_shape=jax.ShapeDtypeStruct((_B, 1), jnp.float32),
        compiler_params=pltpu.CompilerParams(
            vmem_limit_bytes=63 * 1024 * 1024),
    )(ef, e1m, oh, sel, bias.reshape(1, 1), W1, b1.reshape(1, _H),
      g1.reshape(1, _H), bt1.reshape(1, _H), Ws, bs.reshape(_L, 1, _H),
      gs.reshape(_L, 1, _H), bts.reshape(_L, 1, _H), Wf, bf.reshape(1, 1))
    return out[:, 0]


# speedup vs baseline: 3.0166x; 3.0166x over previous
import functools

import jax
import jax.numpy as jnp
from jax import lax
from jax.experimental import pallas as pl
from jax.experimental.pallas import tpu as pltpu
from jax.experimental.pallas import tpu_sc as plsc

_M = 26
_V = 100001
_K = 16
_B = 4096
_H = 400
_L = 2
_EPS = 1e-5

_NC = 2
_NS = 16
_NW = _NC * _NS
_RPW = _B * _M // _NW
_CPW = _RPW // 128


def _sc_gather(x3d, offs1, offs2, tblk, o1p):
    mesh = plsc.VectorSubcoreMesh(core_axis_name="c", subcore_axis_name="s")

    @functools.partial(
        pl.kernel,
        out_type=(
            jax.ShapeDtypeStruct((_B * _M, _K), jnp.float32),
            jax.ShapeDtypeStruct((_B * _M, 16), jnp.float32),
        ),
        mesh=mesh,
        compiler_params=pltpu.CompilerParams(
            use_tc_tiling_on_sc=False, needs_layout_passes=False),
        scratch_types=[
            pltpu.VMEM((_CPW, 128), jnp.int32),
            pltpu.VMEM((_CPW, 128), jnp.int32),
            pltpu.VMEM((_CPW, 128), jnp.int32),
            pltpu.VMEM((128,), jnp.int32),
            pltpu.VMEM((_K, 128), jnp.int32),
            pltpu.VMEM((_K * 128, 16), jnp.float32),
            pltpu.VMEM((128, _K), jnp.float32),
            pltpu.VMEM((128, 16), jnp.float32),
            pltpu.SemaphoreType.DMA,
            pltpu.SemaphoreType.DMA,
        ],
    )
    def k(x3d_h, offs1_h, offs2_h, tblk_h, o1p_h, e2_h, e1_h,
          xv, o1v, o2v, bv, iv, gv, rv, e1v, s2, s1):
        w = lax.axis_index("s") * _NC + lax.axis_index("c")
        pltpu.sync_copy(x3d_h.at[w], xv)
        pltpu.sync_copy(offs1_h, o1v)
        pltpu.sync_copy(offs2_h, o2v)
        lane = lax.iota(jnp.int32, 16)

        def chunk(j, carry):
            for t in range(8):
                sl = pl.ds(t * 16, 16)
                bv[sl] = lax.shift_right_logical(
                    xv[j, sl] + o1v[j, sl], 4)
            c1 = pltpu.async_copy(o1p_h.at[bv], e1v, s1)
            cps = []
            for kk in range(_K):
                for t in range(8):
                    sl = pl.ds(t * 16, 16)
                    iv[kk, sl] = lax.shift_right_logical(
                        xv[j, sl] + o2v[j, sl] + kk * _V, 4)
                cps.append(pltpu.async_copy(
                    tblk_h.at[iv.at[kk]], gv.at[pl.ds(kk * 128, 128)], s2))
            for cp in cps:
                cp.wait()
            for kk in range(_K):
                for t in range(8):
                    sl = pl.ds(t * 16, 16)
                    rows = lane + (kk * 128 + t * 16)
                    lanes = jnp.bitwise_and(xv[j, sl] + kk, 15)
                    vals = plsc.load_gather(gv, [rows, lanes])
                    plsc.store_scatter(rv, [lane + t * 16, lane * 0 + kk],
                                       vals)
            c1.wait()
            base = w * _RPW + j * 128
            pltpu.sync_copy(rv, e2_h.at[pl.ds(base, 128)])
            pltpu.sync_copy(e1v, e1_h.at[pl.ds(base, 128)])
            return carry

        lax.fori_loop(0, _CPW, chunk, 0)

    return k(x3d, offs1, offs2, tblk, o1p)


def _tc_body(ef_ref, e1_ref, oh_ref, sel_ref, bias_ref, w1_ref, b1_ref, g1_ref,
             bt1_ref, ws_ref, bs_ref, gs_ref, bts_ref, wf_ref, bf_ref,
             out_ref):
    f32 = jnp.float32
    hi = lax.Precision.DEFAULT
    ef = ef_ref[...]
    fm1 = jnp.sum((e1_ref[...] * oh_ref[...]).astype(f32), axis=1,
                  keepdims=True)
    sel = sel_ref[...]
    s = lax.dot_general(ef, sel, (((1,), (0,)), ((), ())),
                        precision=hi, preferred_element_type=f32)
    t2 = lax.dot_general(ef * ef, sel, (((1,), (0,)), ((), ())),
                         precision=hi, preferred_element_type=f32)
    fm2 = 0.5 * jnp.sum(s * s - t2, axis=1, keepdims=True)
    y = bias_ref[...] + fm1 + fm2

    def bn_relu(h, g, b):
        mu = jnp.mean(h, axis=0, keepdims=True)
        hc = h - mu
        var = jnp.mean(hc * hc, axis=0, keepdims=True)
        return jnp.maximum(hc * lax.rsqrt(var + _EPS) * g + b, 0.0)

    h = lax.dot_general(ef, w1_ref[...], (((1,), (1,)), ((), ())),
                        precision=hi, preferred_element_type=f32)
    h = bn_relu(h + b1_ref[...], g1_ref[...], bt1_ref[...])
    for i in range(_L):
        h = lax.dot_general(h, ws_ref[i], (((1,), (1,)), ((), ())),
                            precision=hi, preferred_element_type=f32)
        h = bn_relu(h + bs_ref[i], gs_ref[i], bts_ref[i])
    ydnn = lax.dot_general(h, wf_ref[...], (((1,), (1,)), ((), ())),
                           precision=hi, preferred_element_type=f32)
    out_ref[...] = y + ydnn + bf_ref[...]


def kernel(x, o1, o2, bias, W1, b1, g1, bt1, Ws, bs, gs, bts, Wf, bf):
    x3d = x.astype(jnp.int32).reshape(_NW, _CPW, 128)
    marr = jnp.arange(_RPW, dtype=jnp.int32) % _M
    offs1 = (marr * _V).reshape(_CPW, 128)
    offs2 = (marr * (_K * _V)).reshape(_CPW, 128)
    tblk = o2.transpose(0, 2, 1).reshape(_M * _K * _V // 16, 16)
    o1p = jnp.pad(o1.reshape(_M * _V), (0, (-_M * _V) % 16)).reshape(-1, 16)
    e2r, e1r = _sc_gather(x3d, offs1, offs2, tblk, o1p)
    ef = e2r.reshape(_B, _M * _K)
    e1m = e1r.reshape(_B, _M * 16).astype(jnp.bfloat16)
    flat = x.astype(jnp.int32) + jnp.arange(_M, dtype=jnp.int32)[None, :] * _V
    oh = (jnp.bitwise_and(flat, 15)[..., None]
          == jnp.arange(16, dtype=jnp.int32)).astype(jnp.bfloat16)
    oh = oh.reshape(_B, _M * 16)

    sel = (jnp.arange(_M * _K, dtype=jnp.int32)[:, None] % _K
           == jnp.arange(_K, dtype=jnp.int32)[None, :]).astype(jnp.float32)
    out = pl.pallas_call(
        _tc_body,
        out_shape=jax.ShapeDtypeStruct((_B, 1), jnp.float32),
        compiler_params=pltpu.CompilerParams(
            vmem_limit_bytes=63 * 1024 * 1024),
    )(ef, e1m, oh, sel, bias.reshape(1, 1), W1, b1.reshape(1, _H),
      g1.reshape(1, _H), bt1.reshape(1, _H), Ws, bs.reshape(_L, 1, _H),
      gs.reshape(_L, 1, _H), bts.reshape(_L, 1, _H), Wf, bf.reshape(1, 1))
    return out[:, 0]


# speedup vs baseline: 13.5436x; 4.4897x over previous
import functools

import jax
import jax.numpy as jnp
from jax import lax
from jax.experimental import pallas as pl
from jax.experimental.pallas import tpu as pltpu
from jax.experimental.pallas import tpu_sc as plsc

_M = 26
_V = 100001
_K = 16
_B = 4096
_H = 400
_L = 2
_EPS = 1e-5

_NC = 2
_NS = 16
_NW = _NC * _NS
_RPW = _B * _M // _NW
_CPW = _RPW // 128


_VP = 100352
_BPR = _VP // 16


def _detile_body(in_ref, out_ref):
    out_ref[...] = in_ref[...].reshape(_K, _VP // 256, 128)


def _detile(o2p):
    out = pl.pallas_call(
        _detile_body,
        grid=(_M, 2),
        in_specs=[pl.BlockSpec((1, _K, _VP // 2), lambda m, c: (m, 0, c))],
        out_specs=pl.BlockSpec((_K, _VP // 256, 128),
                               lambda m, c: (m, c, 0)),
        out_shape=jax.ShapeDtypeStruct((_M * _K, 784, 128), jnp.float32),
    )(o2p)
    return out.reshape(_M * _K * _BPR, 16)


def _sc_gather(x3d, offs1, offs2, tblk, o1p):
    mesh = plsc.VectorSubcoreMesh(core_axis_name="c", subcore_axis_name="s")

    @functools.partial(
        pl.kernel,
        out_type=(
            jax.ShapeDtypeStruct((_B * _M, _K), jnp.float32),
            jax.ShapeDtypeStruct((_B * _M, 16), jnp.float32),
        ),
        mesh=mesh,
        compiler_params=pltpu.CompilerParams(
            use_tc_tiling_on_sc=False, needs_layout_passes=False),
        scratch_types=[
            pltpu.VMEM((_CPW, 128), jnp.int32),
            pltpu.VMEM((_CPW, 128), jnp.int32),
            pltpu.VMEM((_CPW, 128), jnp.int32),
            pltpu.VMEM((128,), jnp.int32),
            pltpu.VMEM((_K, 128), jnp.int32),
            pltpu.VMEM((_K * 128, 16), jnp.float32),
            pltpu.VMEM((128, _K), jnp.float32),
            pltpu.VMEM((128, 16), jnp.float32),
            pltpu.SemaphoreType.DMA,
            pltpu.SemaphoreType.DMA,
        ],
    )
    def k(x3d_h, offs1_h, offs2_h, tblk_h, o1p_h, e2_h, e1_h,
          xv, o1v, o2v, bv, iv, gv, rv, e1v, s2, s1):
        w = lax.axis_index("s") * _NC + lax.axis_index("c")
        pltpu.sync_copy(x3d_h.at[w], xv)
        pltpu.sync_copy(offs1_h, o1v)
        pltpu.sync_copy(offs2_h, o2v)
        lane = lax.iota(jnp.int32, 16)

        def chunk(j, carry):
            for t in range(8):
                sl = pl.ds(t * 16, 16)
                bv[sl] = lax.shift_right_logical(
                    xv[j, sl] + o1v[j, sl], 4)
            c1 = pltpu.async_copy(o1p_h.at[bv], e1v, s1)
            cps = []
            for kk in range(_K):
                for t in range(8):
                    sl = pl.ds(t * 16, 16)
                    iv[kk, sl] = (o2v[j, sl] + kk * _BPR
                                  + lax.shift_right_logical(xv[j, sl], 4))
                cps.append(pltpu.async_copy(
                    tblk_h.at[iv.at[kk]], gv.at[pl.ds(kk * 128, 128)], s2))
            for cp in cps:
                cp.wait()
            for kk in range(_K):
                for t in range(8):
                    sl = pl.ds(t * 16, 16)
                    rows = lane + (kk * 128 + t * 16)
                    lanes = jnp.bitwise_and(xv[j, sl], 15)
                    vals = plsc.load_gather(gv, [rows, lanes])
                    plsc.store_scatter(rv, [lane + t * 16, lane * 0 + kk],
                                       vals)
            c1.wait()
            base = w * _RPW + j * 128
            pltpu.sync_copy(rv, e2_h.at[pl.ds(base, 128)])
            pltpu.sync_copy(e1v, e1_h.at[pl.ds(base, 128)])
            return carry

        lax.fori_loop(0, _CPW, chunk, 0)

    return k(x3d, offs1, offs2, tblk, o1p)


def _tc_body(ef_ref, e1_ref, oh_ref, sel_ref, bias_ref, w1_ref, b1_ref, g1_ref,
             bt1_ref, ws_ref, bs_ref, gs_ref, bts_ref, wf_ref, bf_ref,
             out_ref):
    f32 = jnp.float32
    hi = lax.Precision.DEFAULT
    ef = ef_ref[...]
    fm1 = jnp.sum((e1_ref[...] * oh_ref[...]).astype(f32), axis=1,
                  keepdims=True)
    sel = sel_ref[...]
    s = lax.dot_general(ef, sel, (((1,), (0,)), ((), ())),
                        precision=hi, preferred_element_type=f32)
    t2 = lax.dot_general(ef * ef, sel, (((1,), (0,)), ((), ())),
                         precision=hi, preferred_element_type=f32)
    fm2 = 0.5 * jnp.sum(s * s - t2, axis=1, keepdims=True)
    y = bias_ref[...] + fm1 + fm2

    def bn_relu(h, g, b):
        mu = jnp.mean(h, axis=0, keepdims=True)
        hc = h - mu
        var = jnp.mean(hc * hc, axis=0, keepdims=True)
        return jnp.maximum(hc * lax.rsqrt(var + _EPS) * g + b, 0.0)

    h = lax.dot_general(ef, w1_ref[...], (((1,), (1,)), ((), ())),
                        precision=hi, preferred_element_type=f32)
    h = bn_relu(h + b1_ref[...], g1_ref[...], bt1_ref[...])
    for i in range(_L):
        h = lax.dot_general(h, ws_ref[i], (((1,), (1,)), ((), ())),
                            precision=hi, preferred_element_type=f32)
        h = bn_relu(h + bs_ref[i], gs_ref[i], bts_ref[i])
    ydnn = lax.dot_general(h, wf_ref[...], (((1,), (1,)), ((), ())),
                           precision=hi, preferred_element_type=f32)
    out_ref[...] = y + ydnn + bf_ref[...]


def kernel(x, o1, o2, bias, W1, b1, g1, bt1, Ws, bs, gs, bts, Wf, bf):
    x3d = x.astype(jnp.int32).reshape(_NW, _CPW, 128)
    marr = jnp.arange(_RPW, dtype=jnp.int32) % _M
    offs1 = (marr * _V).reshape(_CPW, 128)
    offs2 = (marr * (_K * _BPR)).reshape(_CPW, 128)
    tblk = _detile(o2.transpose(0, 2, 1))
    o1p = jnp.pad(o1.reshape(_M * _V), (0, (-_M * _V) % 16)).reshape(-1, 16)
    e2r, e1r = _sc_gather(x3d, offs1, offs2, tblk, o1p)
    ef = e2r.reshape(_B, _M * _K)
    e1m = e1r.reshape(_B, _M * 16).astype(jnp.bfloat16)
    flat = x.astype(jnp.int32) + jnp.arange(_M, dtype=jnp.int32)[None, :] * _V
    oh = (jnp.bitwise_and(flat, 15)[..., None]
          == jnp.arange(16, dtype=jnp.int32)).astype(jnp.bfloat16)
    oh = oh.reshape(_B, _M * 16)

    sel = (jnp.arange(_M * _K, dtype=jnp.int32)[:, None] % _K
           == jnp.arange(_K, dtype=jnp.int32)[None, :]).astype(jnp.float32)
    out = pl.pallas_call(
        _tc_body,
        out_shape=jax.ShapeDtypeStruct((_B, 1), jnp.float32),
        compiler_params=pltpu.CompilerParams(
            vmem_limit_bytes=63 * 1024 * 1024),
    )(ef, e1m, oh, sel, bias.reshape(1, 1), W1, b1.reshape(1, _H),
      g1.reshape(1, _H), bt1.reshape(1, _H), Ws, bs.reshape(_L, 1, _H),
      gs.reshape(_L, 1, _H), bts.reshape(_L, 1, _H), Wf, bf.reshape(1, 1))
    return out[:, 0]
